# baseline (speedup 1.0000x reference)
import functools

import jax
import jax.numpy as jnp
from jax.experimental import pallas as pl
from jax.experimental.pallas import tpu as pltpu

_EMPHASIS = 1.5
_TOPK_PCT = 0.2


def _fused_body(x_ref, w_ref, b_ref, s_ref, emph_ref, *, k):
    x = x_ref[0]
    w = w_ref[...]
    bias = b_ref[0]

    scores = jax.lax.dot_general(
        w, x, (((1,), (1,)), ((), ())),
        preferred_element_type=jnp.float32)
    e = jnp.tanh(scores + bias)

    p = jnp.exp(e)
    z = jnp.sum(p, axis=1, keepdims=True)
    a = p * (1.0 / z)

    ai = jax.lax.bitcast_convert_type(a, jnp.int32)
    j8 = jax.lax.broadcasted_iota(jnp.int32, (8, 1), 0)

    def round3(r, prefix):
        shift = 27 - 3 * r
        cand = prefix | (j8 << shift)
        cnt = jnp.sum((ai >= cand).astype(jnp.int32), axis=1, keepdims=True)
        best = jnp.max(jnp.where(cnt >= k, cand, 0), axis=0, keepdims=True)
        return jnp.broadcast_to(best, (8, 1))

    prefix = jax.lax.fori_loop(0, 10, round3, jnp.zeros((8, 1), jnp.int32),
                               unroll=True)
    kth = prefix[0:1]

    emph = jnp.where(ai >= kth, a * _EMPHASIS, a)
    emph_ref[0] = emph

    s_ref[0] = jax.lax.dot_general(
        emph, x, (((1,), (0,)), ((), ())),
        preferred_element_type=jnp.float32)


@jax.jit
def kernel(x, W, b):
    B, T, D = x.shape
    k = max(int(T * _TOPK_PCT), 1)
    w_row = W.reshape(1, D)
    body = functools.partial(_fused_body, k=k)
    summed, emph = pl.pallas_call(
        body,
        grid=(B,),
        in_specs=[
            pl.BlockSpec((1, T, D), lambda b_: (b_, 0, 0)),
            pl.BlockSpec((1, D), lambda b_: (0, 0)),
            pl.BlockSpec(memory_space=pltpu.SMEM),
        ],
        out_specs=[
            pl.BlockSpec((1, 1, D), lambda b_: (b_, 0, 0)),
            pl.BlockSpec((1, 1, T), lambda b_: (b_, 0, 0)),
        ],
        out_shape=[
            jax.ShapeDtypeStruct((B, 1, D), jnp.float32),
            jax.ShapeDtypeStruct((B, 1, T), jnp.float32),
        ],
        compiler_params=pltpu.CompilerParams(
            dimension_semantics=("arbitrary",),
        ),
    )(x, w_row, b)
    return (summed.reshape(B, D), emph.reshape(B, T))
